# baseline (speedup 1.0000x reference)
import functools

import jax
import jax.numpy as jnp
from jax import lax
from jax.experimental import pallas as pl
from jax.experimental.pallas import tpu as pltpu
from jax.experimental.pallas import tpu_sc as plsc

_INPUT_DIM = 100000
_D = 32
_BATCH = 4096
_SEQ = 200
_N = _BATCH * _SEQ


def _make_kernel():
    info = plsc.get_sparse_core_info()
    nc, ns = info.num_cores, info.num_subcores
    nw = nc * ns
    n_per_w = _N // nw
    gs = 4
    rows = gs * 128
    ng = _SEQ // gs

    mesh = plsc.VectorSubcoreMesh(core_axis_name="c", subcore_axis_name="s")

    @functools.partial(
        pl.kernel,
        mesh=mesh,
        out_type=jax.ShapeDtypeStruct((_SEQ, 4, 32, 8, 128), jnp.float32),
        scratch_types=[
            pltpu.VMEM((n_per_w,), jnp.int32),
            pltpu.VMEM((n_per_w,), jnp.int32),
            pltpu.VMEM((2, rows, _D), jnp.float32),
            pltpu.VMEM((2, gs, 4, 8, 129), jnp.float32),
            pltpu.SemaphoreType.DMA,
            pltpu.SemaphoreType.DMA,
            pltpu.SemaphoreType.DMA,
            pltpu.SemaphoreType.DMA,
        ],
        compiler_params=pltpu.CompilerParams(
            use_tc_tiling_on_sc=False, needs_layout_passes=False
        ),
    )
    def k(idx_hbm, table_hbm, out_hbm, idx_v, gidx_v, gbuf, tbuf,
          g0, g1, o0, o1):
        gsem = (g0, g1)
        osem = (o0, o1)
        w = lax.axis_index("s") * nc + lax.axis_index("c")
        base = w * n_per_w
        pltpu.sync_copy(idx_hbm.at[pl.ds(base, n_per_w)], idx_v)

        iota = lax.iota(jnp.int32, 16)
        iota200 = iota * _SEQ
        zeros = jnp.zeros((16,), jnp.int32)
        dgv = (iota >> 3, (iota >> 3) + 2)
        ddv = iota & 7

        def build_body(s, carry):
            for q in range(8):
                inds = iota200 + (q * 16 * _SEQ + s)
                v = plsc.load_gather(idx_v, [inds])
                v = (
                    jnp.minimum(jnp.maximum(v, -_INPUT_DIM), _INPUT_DIM)
                    + _INPUT_DIM
                )
                gidx_v[pl.ds(s * 128 + q * 16, 16)] = v
            return carry

        lax.fori_loop(0, _SEQ, build_body, 0)

        def gather_desc(i, p):
            return pltpu.make_async_copy(
                table_hbm.at[gidx_v.at[pl.ds(i * rows, rows)]],
                gbuf.at[p],
                gsem[p],
            )

        def out_desc(i, p):
            return pltpu.make_async_copy(
                tbuf.at[p, :, :, :, pl.ds(0, 128)],
                out_hbm.at[pl.ds(i * gs, gs), :, w],
                osem[p],
            )

        def transpose_chunk(p):
            def u_body(j, carry):
                for uu in range(2):
                    u = j * 2 + uu
                    sl = u >> 3
                    q = u & 7
                    r0 = u * 16
                    tb = tbuf.at[p, sl]
                    for kk in range(16):
                        r = r0 + kk
                        bl_s = zeros + (q * 16 + kk)
                        for h in range(2):
                            v = gbuf[p, r, pl.ds(16 * h, 16)]
                            plsc.store_scatter(tb, [dgv[h], ddv, bl_s], v)
                return carry

            lax.fori_loop(0, rows // 32, u_body, 0)

        def step(i, p, wait_prev, issue_next):
            gather_desc(i, p).wait()
            if wait_prev:
                out_desc(i - 2, p).wait()
            transpose_chunk(p)
            out_desc(i, p).start()
            if issue_next:
                gather_desc(i + 2, p).start()

        gather_desc(0, 0).start()
        gather_desc(1, 1).start()
        step(0, 0, False, True)
        step(1, 1, False, True)

        def loop_body(j, carry):
            step(2 * j, 0, True, True)
            step(2 * j + 1, 1, True, True)
            return carry

        lax.fori_loop(1, ng // 2 - 1, loop_body, 0)

        step(ng - 2, 0, True, False)
        step(ng - 1, 1, True, False)
        out_desc(ng - 2, 0).wait()
        out_desc(ng - 1, 1).wait()

    return k


_gather_kernel = _make_kernel()


def kernel(inputs, embeddings):
    flat = inputs.reshape(_N)
    out5 = _gather_kernel(flat, embeddings)
    return out5.transpose(2, 4, 0, 1, 3).reshape(_BATCH, _SEQ, _D)


# speedup vs baseline: 1.0039x; 1.0039x over previous
import functools

import jax
import jax.numpy as jnp
from jax import lax
from jax.experimental import pallas as pl
from jax.experimental.pallas import tpu as pltpu
from jax.experimental.pallas import tpu_sc as plsc

_INPUT_DIM = 100000
_D = 32
_BATCH = 4096
_SEQ = 200
_N = _BATCH * _SEQ


def _make_kernel():
    info = plsc.get_sparse_core_info()
    nc, ns = info.num_cores, info.num_subcores
    nw = nc * ns
    n_per_w = _N // nw
    gs = 4
    rows = gs * 128
    ng = _SEQ // gs

    mesh = plsc.VectorSubcoreMesh(core_axis_name="c", subcore_axis_name="s")

    @functools.partial(
        pl.kernel,
        mesh=mesh,
        out_type=jax.ShapeDtypeStruct((_SEQ, 4, 32, 8, 128), jnp.float32),
        scratch_types=[
            pltpu.VMEM((n_per_w,), jnp.int32),
            pltpu.VMEM((n_per_w,), jnp.int32),
            pltpu.VMEM((2, rows, _D), jnp.float32),
            pltpu.VMEM((2, gs, 4, 8, 129), jnp.float32),
            pltpu.SemaphoreType.DMA,
            pltpu.SemaphoreType.DMA,
            pltpu.SemaphoreType.DMA,
            pltpu.SemaphoreType.DMA,
        ],
        compiler_params=pltpu.CompilerParams(
            use_tc_tiling_on_sc=False, needs_layout_passes=False
        ),
    )
    def k(idx_hbm, table_hbm, out_hbm, idx_v, gidx_v, gbuf, tbuf,
          g0, g1, o0, o1):
        gsem = (g0, g1)
        osem = (o0, o1)
        w = lax.axis_index("s") * nc + lax.axis_index("c")
        base = w * n_per_w
        pltpu.sync_copy(idx_hbm.at[pl.ds(base, n_per_w)], idx_v)

        iota = lax.iota(jnp.int32, 16)
        iota200 = iota * _SEQ
        zeros = jnp.zeros((16,), jnp.int32)
        dgv = (iota >> 3, (iota >> 3) + 2)
        ddv = iota & 7

        def build_body(s, carry):
            for q in range(8):
                inds = iota200 + (q * 16 * _SEQ + s)
                v = plsc.load_gather(idx_v, [inds])
                v = (
                    jnp.minimum(jnp.maximum(v, -_INPUT_DIM), _INPUT_DIM)
                    + _INPUT_DIM
                )
                gidx_v[pl.ds(s * 128 + q * 16, 16)] = v
            return carry

        lax.fori_loop(0, _SEQ, build_body, 0)

        def gather_desc(i, p):
            return pltpu.make_async_copy(
                table_hbm.at[gidx_v.at[pl.ds(i * rows, rows)]],
                gbuf.at[p],
                gsem[p],
            )

        def out_desc(i, p):
            return pltpu.make_async_copy(
                tbuf.at[p, :, :, :, pl.ds(0, 128)],
                out_hbm.at[pl.ds(i * gs, gs), :, w],
                osem[p],
            )

        def transpose_chunk(p):
            def u_body(u, carry):
                sl = u >> 3
                q = u & 7
                r0 = u * 16
                tb = tbuf.at[p, sl]
                for kk in range(16):
                    r = r0 + kk
                    bl_s = zeros + (q * 16 + kk)
                    for h in range(2):
                        v = gbuf[p, r, pl.ds(16 * h, 16)]
                        plsc.store_scatter(tb, [dgv[h], ddv, bl_s], v)
                return carry

            lax.fori_loop(0, rows // 16, u_body, 0)

        def step(i, p, wait_prev, issue_next):
            gather_desc(i, p).wait()
            if wait_prev:
                out_desc(i - 2, p).wait()
            transpose_chunk(p)
            out_desc(i, p).start()
            if issue_next:
                gather_desc(i + 2, p).start()

        gather_desc(0, 0).start()
        gather_desc(1, 1).start()
        step(0, 0, False, True)
        step(1, 1, False, True)

        def loop_body(j, carry):
            step(2 * j, 0, True, True)
            step(2 * j + 1, 1, True, True)
            return carry

        lax.fori_loop(1, ng // 2 - 1, loop_body, 0)

        step(ng - 2, 0, True, False)
        step(ng - 1, 1, True, False)
        out_desc(ng - 2, 0).wait()
        out_desc(ng - 1, 1).wait()

    return k


_gather_kernel = _make_kernel()


def kernel(inputs, embeddings):
    flat = inputs.reshape(_N)
    out5 = _gather_kernel(flat, embeddings)
    return out5.transpose(2, 4, 0, 1, 3).reshape(_BATCH, _SEQ, _D)
